# baseline (speedup 1.0000x reference)
import functools

import jax
import jax.numpy as jnp
from jax import lax
from jax.experimental import pallas as pl
from jax.experimental.pallas import tpu as pltpu
from jax.experimental.pallas import tpu_sc as plsc

NC = 2
NS = 16
NW = NC * NS
L = 16
CH = 128
HD = 64


def _halves(acc_ref):
    av = acc_ref[...]
    return jnp.concatenate([av[0, 0] + av[1, 0], av[0, 1] + av[1, 1]], axis=1)


def _proj_first(x_ref, w_ref, s_ref, d_ref, hlo_ref, hhi_ref,
                hs_ref, hd_ref):
    h = jnp.dot(x_ref[...], w_ref[...], preferred_element_type=jnp.float32)
    hlo_ref[...] = h[:, :HD]
    hhi_ref[...] = h[:, HD:]
    hs_ref[...] = jnp.dot(h, s_ref[...], preferred_element_type=jnp.float32)
    hd_ref[...] = jnp.dot(h, d_ref[...], preferred_element_type=jnp.float32)


def _proj_mid(acc_ref, den_ref, b_ref, w_ref, s_ref, d_ref,
              hlo_ref, hhi_ref, hs_ref, hd_ref):
    den = jnp.sum(den_ref[...], axis=1)
    a = _halves(acc_ref)
    xg = a * (1.0 / den)[:, None] + b_ref[...]
    xe = jnp.where(xg > 0, xg, jnp.exp(xg) - 1.0)
    h = jnp.dot(xe, w_ref[...], preferred_element_type=jnp.float32)
    hlo_ref[...] = h[:, :HD]
    hhi_ref[...] = h[:, HD:]
    hs_ref[...] = jnp.dot(h, s_ref[...], preferred_element_type=jnp.float32)
    hd_ref[...] = jnp.dot(h, d_ref[...], preferred_element_type=jnp.float32)


def _pool(acc_ref, den_ref, b_ref, o_ref, *, n):
    den = jnp.sum(den_ref[...], axis=1)
    a = _halves(acc_ref)
    xg = a * (1.0 / den)[:, None] + b_ref[...]
    xe = jnp.where(xg > 0, xg, jnp.exp(xg) - 1.0)
    s = jnp.sum(xe, axis=0, keepdims=True) * (1.0 / n)

    @pl.when(pl.program_id(0) == 0)
    def _():
        o_ref[...] = jnp.zeros_like(o_ref)

    o_ref[...] += s


def _edge_body(n, n_acc, d, e_true, cpw, hlo_hbm, hhi_hbm, as_hbm, ad_hbm,
               src_hbm, dst_hbm, acc_hbm, den_hbm, as_v, ad_v, den_v, src_m,
               dst_m, ex_m, rows0, rows1, rows2, acc_sh, gs0, gs1, gs2,
               ss0, ss1, ss2):
    rows = (rows0, rows1, rows2)
    gs = (gs0, gs1, gs2)
    ss = (ss0, ss1, ss2)
    cid = lax.axis_index("c")
    sid = lax.axis_index("s")
    wid = sid * NC + cid
    rows_per_tile = n_acc // NS
    slab = CH

    pltpu.sync_copy(as_hbm, as_v)
    pltpu.sync_copy(ad_hbm, ad_v)
    pltpu.sync_copy(src_hbm.at[wid], src_m)
    pltpu.sync_copy(dst_hbm.at[wid], dst_m)

    z16 = jnp.zeros((L,), jnp.float32)

    def _zden(i, c):
        den_v[pl.ds(i * L, L)] = z16
        return c

    lax.fori_loop(0, n_acc // L, _zden, 0)

    def _zero_rows0():
        def _zr(i, c):
            for k in range(HD // L):
                rows0[i, pl.ds(k * L, L)] = z16
            return c

        lax.fori_loop(0, CH, _zr, 0)

    def _zero_stripe():
        def _za(t, c):
            pltpu.sync_copy(
                rows0,
                acc_sh.at[pl.ds(sid * rows_per_tile + t * slab, slab)])
            return c

        lax.fori_loop(0, rows_per_tile // slab, _za, 0)

    _zero_rows0()
    _zero_stripe()

    neg = jnp.full((L,), -3e38, jnp.float32)

    def _mx(i, m):
        return (jnp.maximum(m[0], as_v[pl.ds(i * L, L)]),
                jnp.maximum(m[1], ad_v[pl.ds(i * L, L)]))

    ms, md = lax.fori_loop(0, n // L, _mx, (neg, neg))
    msv, mdv = ms[0], md[0]
    for lane in range(1, L):
        msv = jnp.maximum(msv, ms[lane])
        mdv = jnp.maximum(mdv, md[lane])
    mt = msv + mdv
    cbound = jnp.maximum(mt, 0.2 * mt)

    pltpu.async_copy(hlo_hbm.at[src_m.at[0]], rows0, gs0)
    pltpu.async_copy(hlo_hbm.at[src_m.at[1]], rows1, gs1)
    plsc.subcore_barrier()

    base_e = wid * cpw * CH

    def _p1(j, c):
        for i in range(CH // L):
            s16 = src_m[j, pl.ds(i * L, L)]
            d16 = dst_m[j, pl.ds(i * L, L)]
            al = (plsc.load_gather(as_v, [s16]) +
                  plsc.load_gather(ad_v, [d16]))
            al = jnp.maximum(al, 0.2 * al) - cbound
            ex = jnp.exp(al)
            eidx = base_e + j * CH + i * L + lax.iota(jnp.int32, L)
            ex = jnp.where(eidx < e_true, ex, 0.0)
            ex_m[j, pl.ds(i * L, L)] = ex
            plsc.addupdate_scatter(den_v, [d16], ex)
        return c

    lax.fori_loop(0, cpw, _p1, 0)
    pltpu.sync_copy(den_v, den_hbm.at[wid, 0])

    def _scale(j, buf):
        def _sg(g, cc):
            ex16 = ex_m[j, pl.ds(g * L, L)]
            for lane in range(L):
                e = g * L + lane
                coef = ex16[lane]
                for k in range(HD // L):
                    buf[e, pl.ds(k * L, L)] = buf[e, pl.ds(k * L, L)] * coef
            return cc

        lax.fori_loop(0, CH // L, _sg, 0)

    def _run_half(h_hbm):
        def _step(j, b, first, last):
            pltpu.make_async_copy(h_hbm.at[src_m.at[j]], rows[b],
                                  gs[b]).wait()
            _scale(j, rows[b])
            pltpu.async_copy(rows[b], acc_sh.at[dst_m.at[j]], ss[b],
                             add=True)
            if not last:
                nb = (b + 2) % 3
                if not first:
                    pltpu.make_async_copy(h_hbm.at[src_m.at[j]], rows[nb],
                                          ss[nb]).wait()
                pltpu.async_copy(h_hbm.at[src_m.at[j + 2]], rows[nb], gs[nb])

        _step(0, 0, True, False)
        _step(1, 1, False, False)
        _step(2, 2, False, False)

        def _ring(g, c):
            for b in range(3):
                _step(g * 3 + b, b, False, False)
            return c

        lax.fori_loop(1, cpw // 3 - 1, _ring, 0)
        _step(cpw - 3, 0, False, False)
        _step(cpw - 2, 1, False, True)
        _step(cpw - 1, 2, False, True)
        for b in range(3):
            pltpu.make_async_copy(h_hbm.at[src_m.at[0]], rows[b],
                                  ss[b]).wait()
        plsc.subcore_barrier()

    def _write_stripe(half):
        def _wa(t, c):
            r0 = sid * rows_per_tile + t * slab
            pltpu.sync_copy(acc_sh.at[pl.ds(r0, slab)],
                            acc_hbm.at[cid, half, pl.ds(r0, slab)])
            return c

        lax.fori_loop(0, rows_per_tile // slab, _wa, 0)

    _run_half(hlo_hbm)
    _write_stripe(0)
    _zero_rows0()
    _zero_stripe()
    pltpu.async_copy(hhi_hbm.at[src_m.at[0]], rows0, gs0)
    pltpu.async_copy(hhi_hbm.at[src_m.at[1]], rows1, gs1)
    plsc.subcore_barrier()
    _run_half(hhi_hbm)
    _write_stripe(1)


@functools.lru_cache(maxsize=None)
def _build(n, d, e):
    e_true = e + n
    cpw = -(-e_true // (NW * CH))
    cpw = -(-cpw // 3) * 3
    assert cpw >= 6
    e_pad = NW * cpw * CH
    n_acc = -(-n // (NS * CH)) * NS * CH
    r = 1000
    grid = n // r
    f32 = jnp.float32

    h_out_specs = [
        pl.BlockSpec((r, HD), lambda i: (i, 0)),
        pl.BlockSpec((r, HD), lambda i: (i, 0)),
        pl.BlockSpec((r, 1), lambda i: (i, 0)),
        pl.BlockSpec((r, 1), lambda i: (i, 0)),
    ]
    h_out_shape = [
        jax.ShapeDtypeStruct((n, HD), f32),
        jax.ShapeDtypeStruct((n, HD), f32),
        jax.ShapeDtypeStruct((n, 1), f32),
        jax.ShapeDtypeStruct((n, 1), f32),
    ]

    proj_first = pl.pallas_call(
        _proj_first,
        grid=(grid,),
        in_specs=[
            pl.BlockSpec((r, d), lambda i: (i, 0)),
            pl.BlockSpec((d, d), lambda i: (0, 0)),
            pl.BlockSpec((d, 1), lambda i: (0, 0)),
            pl.BlockSpec((d, 1), lambda i: (0, 0)),
        ],
        out_specs=h_out_specs,
        out_shape=h_out_shape,
    )

    proj_mid = pl.pallas_call(
        _proj_mid,
        grid=(grid,),
        in_specs=[
            pl.BlockSpec((NC, 2, r, HD), lambda i: (0, 0, i, 0)),
            pl.BlockSpec((r, NW), lambda i: (i, 0)),
            pl.BlockSpec((1, d), lambda i: (0, 0)),
            pl.BlockSpec((d, d), lambda i: (0, 0)),
            pl.BlockSpec((d, 1), lambda i: (0, 0)),
            pl.BlockSpec((d, 1), lambda i: (0, 0)),
        ],
        out_specs=h_out_specs,
        out_shape=h_out_shape,
    )

    pool = pl.pallas_call(
        functools.partial(_pool, n=n),
        grid=(grid,),
        in_specs=[
            pl.BlockSpec((NC, 2, r, HD), lambda i: (0, 0, i, 0)),
            pl.BlockSpec((r, NW), lambda i: (i, 0)),
            pl.BlockSpec((1, d), lambda i: (0, 0)),
        ],
        out_specs=pl.BlockSpec((1, d), lambda i: (0, 0)),
        out_shape=jax.ShapeDtypeStruct((1, d), f32),
    )

    mesh = plsc.VectorSubcoreMesh(core_axis_name="c", subcore_axis_name="s")
    edge_call = pl.kernel(
        functools.partial(_edge_body, n, n_acc, d, e_true, cpw),
        out_type=[
            jax.ShapeDtypeStruct((NC, 2, n_acc, HD), f32),
            jax.ShapeDtypeStruct((NW, 1, n_acc), f32),
        ],
        mesh=mesh,
        scratch_types=[
            pltpu.VMEM((n,), f32),
            pltpu.VMEM((n,), f32),
            pltpu.VMEM((n_acc,), f32),
            pltpu.VMEM((cpw, CH), jnp.int32),
            pltpu.VMEM((cpw, CH), jnp.int32),
            pltpu.VMEM((cpw, CH), f32),
            pltpu.VMEM((CH, HD), f32),
            pltpu.VMEM((CH, HD), f32),
            pltpu.VMEM((CH, HD), f32),
            pltpu.VMEM_SHARED((n_acc, HD), f32),
            pltpu.SemaphoreType.DMA,
            pltpu.SemaphoreType.DMA,
            pltpu.SemaphoreType.DMA,
            pltpu.SemaphoreType.DMA,
            pltpu.SemaphoreType.DMA,
            pltpu.SemaphoreType.DMA,
        ],
        compiler_params=pltpu.CompilerParams(needs_layout_passes=False,
                                             use_tc_tiling_on_sc=False),
    )
    return proj_first, proj_mid, pool, edge_call, e_pad, e_true, n_acc


def kernel(x, edge_index, W1, att_src1, att_dst1, bias1,
           W2, att_src2, att_dst2, bias2,
           W3, att_src3, att_dst3, bias3):
    n, d = x.shape
    e = edge_index.shape[1]
    (proj_first, proj_mid, pool, edge_call,
     e_pad, e_true, n_acc) = _build(n, d, e)

    loop = jnp.arange(n, dtype=jnp.int32)
    pad = jnp.zeros((e_pad - e_true,), jnp.int32)
    src = jnp.concatenate([edge_index[0], loop, pad]).reshape(NW, -1, CH)
    dst = jnp.concatenate([edge_index[1], loop, pad]).reshape(NW, -1, CH)

    hlo, hhi, hs, hd = proj_first(x, W1, att_src1.reshape(d, 1),
                                  att_dst1.reshape(d, 1))
    acc, den = edge_call(hlo, hhi, hs.reshape(n), hd.reshape(n), src, dst)

    hlo, hhi, hs, hd = proj_mid(
        acc, den.reshape(NW, n_acc).T, bias1.reshape(1, d),
        W2, att_src2.reshape(d, 1), att_dst2.reshape(d, 1))
    acc, den = edge_call(hlo, hhi, hs.reshape(n), hd.reshape(n), src, dst)

    hlo, hhi, hs, hd = proj_mid(
        acc, den.reshape(NW, n_acc).T, bias2.reshape(1, d),
        W3, att_src3.reshape(d, 1), att_dst3.reshape(d, 1))
    acc, den = edge_call(hlo, hhi, hs.reshape(n), hd.reshape(n), src, dst)

    return pool(acc, den.reshape(NW, n_acc).T, bias3.reshape(1, d))


# speedup vs baseline: 1.3398x; 1.3398x over previous
import functools

import jax
import jax.numpy as jnp
from jax import lax
from jax.experimental import pallas as pl
from jax.experimental.pallas import tpu as pltpu
from jax.experimental.pallas import tpu_sc as plsc

NC = 2
NS = 16
NW = NC * NS
L = 16
CHS = 64


def _proj_first(x_ref, w_ref, s_ref, d_ref, h_ref, hs_ref, hd_ref):
    h = jnp.dot(x_ref[...], w_ref[...], preferred_element_type=jnp.float32)
    h_ref[...] = h
    hs_ref[...] = jnp.dot(h, s_ref[...], preferred_element_type=jnp.float32)
    hd_ref[...] = jnp.dot(h, d_ref[...], preferred_element_type=jnp.float32)


def _proj_mid(acc_ref, den_ref, b_ref, w_ref, s_ref, d_ref,
              h_ref, hs_ref, hd_ref):
    den = jnp.sum(den_ref[...], axis=1)
    a = acc_ref[0] + acc_ref[1]
    xg = a * (1.0 / den)[:, None] + b_ref[...]
    xe = jnp.where(xg > 0, xg, jnp.exp(xg) - 1.0)
    h = jnp.dot(xe, w_ref[...], preferred_element_type=jnp.float32)
    h_ref[...] = h
    hs_ref[...] = jnp.dot(h, s_ref[...], preferred_element_type=jnp.float32)
    hd_ref[...] = jnp.dot(h, d_ref[...], preferred_element_type=jnp.float32)


def _pool(acc_ref, den_ref, b_ref, o_ref, *, n):
    den = jnp.sum(den_ref[...], axis=1)
    a = acc_ref[0] + acc_ref[1]
    xg = a * (1.0 / den)[:, None] + b_ref[...]
    xe = jnp.where(xg > 0, xg, jnp.exp(xg) - 1.0)
    s = jnp.sum(xe, axis=0, keepdims=True) * (1.0 / n)

    @pl.when(pl.program_id(0) == 0)
    def _():
        o_ref[...] = jnp.zeros_like(o_ref)

    o_ref[...] += s


def _edge_body(n, n_acc, d, e_true, cpw, h_hbm, as_hbm, ad_hbm, src_hbm,
               dst_hbm, acc_hbm, den_hbm, as_v, ad_v, den_v,
               sb0, sb1, sb2, sb3, db0, db1, db2, db3, exb0, exb1,
               rows0, rows1, acc_sh,
               is0, is1, is2, is3, gs0, gs1, ss0, ss1):
    sb = (sb0, sb1, sb2, sb3)
    db = (db0, db1, db2, db3)
    isem = (is0, is1, is2, is3)
    exb = (exb0, exb1)
    rows = (rows0, rows1)
    gs = (gs0, gs1)
    ss = (ss0, ss1)
    cid = lax.axis_index("c")
    sid = lax.axis_index("s")
    wid = sid * NC + cid
    rows_per_tile = n_acc // NS

    pltpu.sync_copy(as_hbm, as_v)
    pltpu.sync_copy(ad_hbm, ad_v)

    z16 = jnp.zeros((L,), jnp.float32)

    def _zden(i, c):
        den_v[pl.ds(i * L, L)] = z16
        return c

    lax.fori_loop(0, n_acc // L, _zden, 0)

    def _zrows(i, c):
        for k in range(d // L):
            rows0[i, pl.ds(k * L, L)] = z16
        return c

    lax.fori_loop(0, CHS, _zrows, 0)

    def _zacc(t, c):
        pltpu.sync_copy(rows0,
                        acc_sh.at[pl.ds(sid * rows_per_tile + t * CHS, CHS)])
        return c

    lax.fori_loop(0, rows_per_tile // CHS, _zacc, 0)

    neg = jnp.full((L,), -3e38, jnp.float32)

    def _mx(i, m):
        return (jnp.maximum(m[0], as_v[pl.ds(i * L, L)]),
                jnp.maximum(m[1], ad_v[pl.ds(i * L, L)]))

    ms, md = lax.fori_loop(0, n // L, _mx, (neg, neg))
    msv, mdv = ms[0], md[0]
    for lane in range(1, L):
        msv = jnp.maximum(msv, ms[lane])
        mdv = jnp.maximum(mdv, md[lane])
    mt = msv + mdv
    cbound = jnp.maximum(mt, 0.2 * mt)

    base_e = wid * cpw * CHS

    def _issue_idx(j, q):
        pltpu.async_copy(src_hbm.at[wid, j], sb[q], isem[q])
        pltpu.async_copy(dst_hbm.at[wid, j], db[q], isem[q])

    def _wait_idx(j, q):
        pltpu.make_async_copy(src_hbm.at[wid, j], sb[q], isem[q]).wait()
        pltpu.make_async_copy(dst_hbm.at[wid, j], db[q], isem[q]).wait()

    def _step(j, b, q, issue_idx, wait_ss, wait_isem, issue_gather):
        pltpu.make_async_copy(h_hbm.at[sb[q].at[0]], rows[b], gs[b]).wait()
        if issue_idx:
            _issue_idx(j + 2, (q + 2) % 4)
        for i in range(CHS // L):
            s16 = sb[q][0, pl.ds(i * L, L)]
            d16 = db[q][0, pl.ds(i * L, L)]
            al = (plsc.load_gather(as_v, [s16]) +
                  plsc.load_gather(ad_v, [d16]))
            al = jnp.maximum(al, 0.2 * al) - cbound
            ex = jnp.exp(al)
            eidx = base_e + j * CHS + i * L + lax.iota(jnp.int32, L)
            ex = jnp.where(eidx < e_true, ex, 0.0)
            exb[b][pl.ds(i * L, L)] = ex
            plsc.addupdate_scatter(den_v, [d16], ex)

        def _srow(g, cc):
            ex16 = exb[b][pl.ds(g * L, L)]
            for lane in range(L):
                e = g * L + lane
                coef = ex16[lane]
                for k in range(d // L):
                    rows[b][e, pl.ds(k * L, L)] = (
                        rows[b][e, pl.ds(k * L, L)] * coef)
            return cc

        lax.fori_loop(0, CHS // L, _srow, 0)
        pltpu.async_copy(rows[b], acc_sh.at[db[q].at[0]], ss[b], add=True)
        if wait_ss:
            pltpu.make_async_copy(h_hbm.at[sb[q].at[0]], rows[1 - b],
                                  ss[1 - b]).wait()
        if issue_gather:
            if wait_isem:
                _wait_idx(j + 1, (q + 1) % 4)
            pltpu.async_copy(h_hbm.at[sb[(q + 1) % 4].at[0]], rows[1 - b],
                             gs[1 - b])

    pltpu.sync_copy(src_hbm.at[wid, 0], sb[0])
    pltpu.sync_copy(dst_hbm.at[wid, 0], db[0])
    pltpu.sync_copy(src_hbm.at[wid, 1], sb[1])
    pltpu.sync_copy(dst_hbm.at[wid, 1], db[1])
    _issue_idx(2, 2)
    _issue_idx(3, 3)
    pltpu.async_copy(h_hbm.at[sb[0].at[0]], rows0, gs0)
    plsc.subcore_barrier()

    _step(0, 0, 0, False, False, False, True)
    _step(1, 1, 1, False, True, True, True)
    _step(2, 0, 2, True, True, True, True)
    _step(3, 1, 3, True, True, True, True)

    def _quad(g, c):
        j0 = 4 + g * 4
        _step(j0 + 0, 0, 0, True, True, True, True)
        _step(j0 + 1, 1, 1, True, True, True, True)
        _step(j0 + 2, 0, 2, True, True, True, True)
        _step(j0 + 3, 1, 3, True, True, True, True)
        return c

    lax.fori_loop(0, (cpw - 6) // 4, _quad, 0)
    _step(cpw - 2, 0, 0, False, True, True, True)
    _step(cpw - 1, 1, 1, False, True, False, False)
    pltpu.make_async_copy(h_hbm.at[sb[0].at[0]], rows1, ss[1]).wait()
    plsc.subcore_barrier()

    def _wacc(t, c):
        r0 = sid * rows_per_tile + t * CHS
        pltpu.sync_copy(acc_sh.at[pl.ds(r0, CHS)],
                        acc_hbm.at[cid, pl.ds(r0, CHS)])
        return c

    lax.fori_loop(0, rows_per_tile // CHS, _wacc, 0)
    pltpu.sync_copy(den_v, den_hbm.at[wid, 0])


@functools.lru_cache(maxsize=None)
def _build(n, d, e):
    e_true = e + n
    cpw = -(-e_true // (NW * CHS))
    cpw += (2 - cpw) % 4
    assert cpw >= 6 and cpw % 4 == 2
    e_pad = NW * cpw * CHS
    n_acc = -(-n // (NS * CHS)) * NS * CHS
    r = 1000
    grid = n // r
    f32 = jnp.float32

    proj_first = pl.pallas_call(
        _proj_first,
        grid=(grid,),
        in_specs=[
            pl.BlockSpec((r, d), lambda i: (i, 0)),
            pl.BlockSpec((d, d), lambda i: (0, 0)),
            pl.BlockSpec((d, 1), lambda i: (0, 0)),
            pl.BlockSpec((d, 1), lambda i: (0, 0)),
        ],
        out_specs=[
            pl.BlockSpec((r, d), lambda i: (i, 0)),
            pl.BlockSpec((r, 1), lambda i: (i, 0)),
            pl.BlockSpec((r, 1), lambda i: (i, 0)),
        ],
        out_shape=[
            jax.ShapeDtypeStruct((n, d), f32),
            jax.ShapeDtypeStruct((n, 1), f32),
            jax.ShapeDtypeStruct((n, 1), f32),
        ],
    )

    proj_mid = pl.pallas_call(
        _proj_mid,
        grid=(grid,),
        in_specs=[
            pl.BlockSpec((NC, r, d), lambda i: (0, i, 0)),
            pl.BlockSpec((r, NW), lambda i: (i, 0)),
            pl.BlockSpec((1, d), lambda i: (0, 0)),
            pl.BlockSpec((d, d), lambda i: (0, 0)),
            pl.BlockSpec((d, 1), lambda i: (0, 0)),
            pl.BlockSpec((d, 1), lambda i: (0, 0)),
        ],
        out_specs=[
            pl.BlockSpec((r, d), lambda i: (i, 0)),
            pl.BlockSpec((r, 1), lambda i: (i, 0)),
            pl.BlockSpec((r, 1), lambda i: (i, 0)),
        ],
        out_shape=[
            jax.ShapeDtypeStruct((n, d), f32),
            jax.ShapeDtypeStruct((n, 1), f32),
            jax.ShapeDtypeStruct((n, 1), f32),
        ],
    )

    pool = pl.pallas_call(
        functools.partial(_pool, n=n),
        grid=(grid,),
        in_specs=[
            pl.BlockSpec((NC, r, d), lambda i: (0, i, 0)),
            pl.BlockSpec((r, NW), lambda i: (i, 0)),
            pl.BlockSpec((1, d), lambda i: (0, 0)),
        ],
        out_specs=pl.BlockSpec((1, d), lambda i: (0, 0)),
        out_shape=jax.ShapeDtypeStruct((1, d), f32),
    )

    mesh = plsc.VectorSubcoreMesh(core_axis_name="c", subcore_axis_name="s")
    edge_call = pl.kernel(
        functools.partial(_edge_body, n, n_acc, d, e_true, cpw),
        out_type=[
            jax.ShapeDtypeStruct((NC, n_acc, d), f32),
            jax.ShapeDtypeStruct((NW, 1, n_acc), f32),
        ],
        mesh=mesh,
        scratch_types=(
            [pltpu.VMEM((n,), f32),
             pltpu.VMEM((n,), f32),
             pltpu.VMEM((n_acc,), f32)]
            + [pltpu.VMEM((1, CHS), jnp.int32) for _ in range(8)]
            + [pltpu.VMEM((CHS,), f32) for _ in range(2)]
            + [pltpu.VMEM((CHS, d), f32) for _ in range(2)]
            + [pltpu.VMEM_SHARED((n_acc, d), f32)]
            + [pltpu.SemaphoreType.DMA for _ in range(8)]
        ),
        compiler_params=pltpu.CompilerParams(needs_layout_passes=False),
    )
    return proj_first, proj_mid, pool, edge_call, e_pad, e_true, n_acc


def kernel(x, edge_index, W1, att_src1, att_dst1, bias1,
           W2, att_src2, att_dst2, bias2,
           W3, att_src3, att_dst3, bias3):
    n, d = x.shape
    e = edge_index.shape[1]
    (proj_first, proj_mid, pool, edge_call,
     e_pad, e_true, n_acc) = _build(n, d, e)

    loop = jnp.arange(n, dtype=jnp.int32)
    pad = jnp.zeros((e_pad - e_true,), jnp.int32)
    src = jnp.concatenate([edge_index[0], loop, pad]).reshape(NW, -1, 1, CHS)
    dst = jnp.concatenate([edge_index[1], loop, pad]).reshape(NW, -1, 1, CHS)

    h, hs, hd = proj_first(x, W1, att_src1.reshape(d, 1),
                           att_dst1.reshape(d, 1))
    acc, den = edge_call(h, hs.reshape(n), hd.reshape(n), src, dst)

    h, hs, hd = proj_mid(acc, den.reshape(NW, n_acc).T, bias1.reshape(1, d),
                         W2, att_src2.reshape(d, 1), att_dst2.reshape(d, 1))
    acc, den = edge_call(h, hs.reshape(n), hd.reshape(n), src, dst)

    h, hs, hd = proj_mid(acc, den.reshape(NW, n_acc).T, bias2.reshape(1, d),
                         W3, att_src3.reshape(d, 1), att_dst3.reshape(d, 1))
    acc, den = edge_call(h, hs.reshape(n), hd.reshape(n), src, dst)

    return pool(acc, den.reshape(NW, n_acc).T, bias3.reshape(1, d))
